# baseline (speedup 1.0000x reference)
import functools

import jax
import jax.numpy as jnp
from jax import lax
from jax.experimental import pallas as pl
from jax.experimental.pallas import tpu as pltpu
from jax.experimental.pallas import tpu_sc as plsc

G = 128
EPS = 1e-4
N = 50000
C = 128
NOFF = 27
NENT = 28

NW = 32
CPP = 16
PPT = 1568
NPAD = NW * PPT
NCHUNK = PPT // CPP
EPC = CPP * NENT
NSEG = 8
SEGLEN = 56

TBL = 102 * G * G + 102


def _stats_kernel(x_ref, o_ref):
    i = pl.program_id(0)

    @pl.when(i == 0)
    def _():
        o_ref[...] = jnp.zeros_like(o_ref)

    xb = x_ref[...].astype(jnp.float32)
    o_ref[0:1, :] += jnp.sum(xb, axis=0, keepdims=True)
    o_ref[1:2, :] += jnp.sum(xb * xb, axis=0, keepdims=True)


def _bn_stats(x, bm=1024):
    nb = x.shape[0] // bm
    return pl.pallas_call(
        _stats_kernel,
        grid=(nb,),
        in_specs=[pl.BlockSpec((bm, C), lambda i: (i, 0))],
        out_specs=pl.BlockSpec((8, C), lambda i: (0, 0)),
        out_shape=jax.ShapeDtypeStruct((8, C), jnp.float32),
    )(x)


def _mm_kernel(x_ref, st_ref, w_ref, *rest, bm, nj, with_res):
    if with_res:
        res_ref, y_ref = rest
    else:
        (y_ref,) = rest
    i = pl.program_id(0)
    s = st_ref[0:1, :]
    s2 = st_ref[1:2, :]
    bw = st_ref[2:3, :]
    bb = st_ref[3:4, :]
    mean = s * (1.0 / N)
    var = s2 * (1.0 / N) - mean * mean
    rstd = lax.rsqrt(var + EPS)
    scale = rstd * bw
    shift = bb - mean * scale

    xb = x_ref[...].astype(jnp.float32)
    h = jnp.maximum(xb * scale + shift, 0.0)
    rows = i * bm + lax.broadcasted_iota(jnp.int32, (bm, 1), 0)
    h = jnp.where(rows < N, h, 0.0).astype(jnp.bfloat16)
    for j in range(nj):
        y_ref[j] = jnp.dot(h, w_ref[j], preferred_element_type=jnp.float32)
    if with_res:
        y_ref[nj] = res_ref[...]


def _bn_relu_mm(x, stats, w_bf, res=None, bm=512):
    nb = x.shape[0] // bm
    nj = w_bf.shape[0]
    nblk = nj + (1 if res is not None else 0)
    in_specs = [
        pl.BlockSpec((bm, C), lambda i: (i, 0)),
        pl.BlockSpec((8, C), lambda i: (0, 0)),
        pl.BlockSpec((nj, C, C), lambda i: (0, 0, 0)),
    ]
    args = [x, stats, w_bf]
    if res is not None:
        in_specs.append(pl.BlockSpec((bm, C), lambda i: (i, 0)))
        args.append(res)
    return pl.pallas_call(
        functools.partial(_mm_kernel, bm=bm, nj=nj, with_res=res is not None),
        grid=(nb,),
        in_specs=in_specs,
        out_specs=pl.BlockSpec((nblk, bm, C), lambda i: (0, i, 0)),
        out_shape=jax.ShapeDtypeStruct((nblk, NPAD, C), jnp.float32),
    )(*args)


def _sc_body(y_hbm, idx_hbm, out_hbm,
             idxv0, idxv1, rows0, rows1, out0, out1,
             sr0, sr1, so0, so1, si0, si1):
    w = lax.axis_index("s") * 2 + lax.axis_index("c")
    base = w * PPT
    idxv = (idxv0, idxv1)
    rows = (rows0, rows1)
    outs = (out0, out1)
    srs = (sr0, sr1)
    sos = (so0, so1)
    sis = (si0, si1)

    def issue(g, b):
        for r in range(NSEG):
            pltpu.async_copy(y_hbm.at[idxv[b].at[r]],
                             rows[b].at[pl.ds(r * SEGLEN, SEGLEN)],
                             srs[b])

    def drain_rows(b):
        pltpu.make_async_copy(y_hbm.at[pl.ds(0, EPC)], rows[b],
                              srs[b]).wait()

    def drain_out(b):
        pltpu.make_async_copy(out_hbm.at[pl.ds(0, CPP)], outs[b],
                              sos[b]).wait()

    def drain_idx(b):
        pltpu.make_async_copy(idx_hbm.at[0, 0], idxv[b], sis[b]).wait()

    pltpu.sync_copy(idx_hbm.at[w, 0], idxv0)
    issue(0, 0)
    pltpu.sync_copy(idx_hbm.at[w, 1], idxv1)
    issue(1, 1)

    def process(gg, b):
        drain_rows(b)

        @pl.when(gg + 2 < NCHUNK)
        def _():
            pltpu.async_copy(idx_hbm.at[w, gg + 2], idxv[b], sis[b])

        @pl.when(gg >= 2)
        def _():
            drain_out(b)

        def point(pp, c2):
            e0 = pp * NENT
            for grp in range(8):
                sl = pl.ds(grp * 16, 16)
                acc = rows[b][e0, sl]
                for j in range(1, NENT):
                    acc = acc + rows[b][e0 + j, sl]
                outs[b][pp, sl] = acc
            return c2

        lax.fori_loop(0, CPP, point, 0)
        pltpu.async_copy(outs[b], out_hbm.at[pl.ds(base + gg * CPP, CPP)],
                         sos[b])

        @pl.when(gg + 2 < NCHUNK)
        def _():
            drain_idx(b)
            issue(gg + 2, b)

    def chunk(g2, c):
        process(g2 * 2, 0)
        process(g2 * 2 + 1, 1)
        return c

    lax.fori_loop(0, NCHUNK // 2, chunk, 0)
    drain_out(0)
    drain_out(1)


def _sc_gathersum(y, idx):
    yf = y.reshape(-1, C)
    mesh = plsc.VectorSubcoreMesh(core_axis_name="c", subcore_axis_name="s")
    k = pl.kernel(
        _sc_body,
        out_type=jax.ShapeDtypeStruct((NPAD, C), jnp.float32),
        mesh=mesh,
        scratch_types=[
            pltpu.VMEM((NSEG, SEGLEN), jnp.int32),
            pltpu.VMEM((NSEG, SEGLEN), jnp.int32),
            pltpu.VMEM((EPC, C), jnp.float32),
            pltpu.VMEM((EPC, C), jnp.float32),
            pltpu.VMEM((CPP, C), jnp.float32),
            pltpu.VMEM((CPP, C), jnp.float32),
            pltpu.SemaphoreType.DMA,
            pltpu.SemaphoreType.DMA,
            pltpu.SemaphoreType.DMA,
            pltpu.SemaphoreType.DMA,
            pltpu.SemaphoreType.DMA,
            pltpu.SemaphoreType.DMA,
        ],
    )
    return k(yf, idx)


def _build_indices(pos):
    p = pos.astype(jnp.int32) + 1
    key = p[:, 0] * (G * G) + p[:, 1] * G + p[:, 2]
    table = jnp.full((TBL,), N, dtype=jnp.int32)
    table = table.at[key].min(jnp.arange(N, dtype=jnp.int32))
    offs = []
    for dx in (-1, 0, 1):
        for dy in (-1, 0, 1):
            for dz in (-1, 0, 1):
                offs.append(dx * G * G + dy * G + dz)
    offs = jnp.array(offs, dtype=jnp.int32)
    nk = key[:, None] + offs[None, :]
    src = jnp.full((NPAD, NOFF), N, dtype=jnp.int32).at[:N].set(table[nk])
    joff = jnp.arange(NOFF, dtype=jnp.int32) * NPAD
    i = jnp.arange(NPAD, dtype=jnp.int32)
    zrow = N + (i[:, None] + jnp.arange(NOFF, dtype=jnp.int32) * 7) \
        % (NPAD - N)
    rowid = jnp.where(src < N, src, zrow) + joff[None, :]
    z28 = (N + (i * 13) % (NPAD - N))[:, None]
    res28 = (NOFF * NPAD + i)[:, None]
    idx1 = jnp.concatenate([rowid, z28], axis=1)
    idx2 = jnp.concatenate([rowid, res28], axis=1)
    shape = (NW, NCHUNK, NSEG, SEGLEN)
    return idx1.reshape(shape), idx2.reshape(shape)


def kernel(feat, pos, W1, W2, bn1_w, bn1_b, bn2_w, bn2_b):
    idx1, idx2 = _build_indices(pos)
    featp = jnp.pad(feat, ((0, NPAD - N), (0, 0)))
    w1_bf = W1.astype(jnp.bfloat16)
    w2_bf = W2.astype(jnp.bfloat16)

    st1 = _bn_stats(featp).at[2].set(bn1_w).at[3].set(bn1_b)
    y1 = _bn_relu_mm(featp, st1, w1_bf)
    c1 = _sc_gathersum(y1, idx1)

    st2 = _bn_stats(c1).at[2].set(bn2_w).at[3].set(bn2_b)
    y2 = _bn_relu_mm(c1, st2, w2_bf, res=featp)
    out = _sc_gathersum(y2, idx2)
    return out[:N]


# speedup vs baseline: 1.0018x; 1.0018x over previous
import functools

import jax
import jax.numpy as jnp
from jax import lax
from jax.experimental import pallas as pl
from jax.experimental.pallas import tpu as pltpu
from jax.experimental.pallas import tpu_sc as plsc

G = 128
EPS = 1e-4
N = 50000
C = 128
NOFF = 27
NENT = 28

NW = 32
CPP = 16
PPT = 1568
NPAD = NW * PPT
NCHUNK = PPT // CPP
EPC = CPP * NENT
NSEG = 4
SEGLEN = 112

TBL = 102 * G * G + 102


def _stats_kernel(x_ref, o_ref):
    i = pl.program_id(0)

    @pl.when(i == 0)
    def _():
        o_ref[...] = jnp.zeros_like(o_ref)

    xb = x_ref[...].astype(jnp.float32)
    o_ref[0:1, :] += jnp.sum(xb, axis=0, keepdims=True)
    o_ref[1:2, :] += jnp.sum(xb * xb, axis=0, keepdims=True)


def _bn_stats(x, bm=1024):
    nb = x.shape[0] // bm
    return pl.pallas_call(
        _stats_kernel,
        grid=(nb,),
        in_specs=[pl.BlockSpec((bm, C), lambda i: (i, 0))],
        out_specs=pl.BlockSpec((8, C), lambda i: (0, 0)),
        out_shape=jax.ShapeDtypeStruct((8, C), jnp.float32),
    )(x)


def _mm_kernel(x_ref, st_ref, w_ref, *rest, bm, nj, with_res):
    if with_res:
        res_ref, y_ref = rest
    else:
        (y_ref,) = rest
    i = pl.program_id(0)
    s = st_ref[0:1, :]
    s2 = st_ref[1:2, :]
    bw = st_ref[2:3, :]
    bb = st_ref[3:4, :]
    mean = s * (1.0 / N)
    var = s2 * (1.0 / N) - mean * mean
    rstd = lax.rsqrt(var + EPS)
    scale = rstd * bw
    shift = bb - mean * scale

    xb = x_ref[...].astype(jnp.float32)
    h = jnp.maximum(xb * scale + shift, 0.0)
    rows = i * bm + lax.broadcasted_iota(jnp.int32, (bm, 1), 0)
    h = jnp.where(rows < N, h, 0.0).astype(jnp.bfloat16)
    for j in range(nj):
        y_ref[j] = jnp.dot(h, w_ref[j], preferred_element_type=jnp.float32)
    if with_res:
        y_ref[nj] = res_ref[...]


def _bn_relu_mm(x, stats, w_bf, res=None, bm=512):
    nb = x.shape[0] // bm
    nj = w_bf.shape[0]
    nblk = nj + (1 if res is not None else 0)
    in_specs = [
        pl.BlockSpec((bm, C), lambda i: (i, 0)),
        pl.BlockSpec((8, C), lambda i: (0, 0)),
        pl.BlockSpec((nj, C, C), lambda i: (0, 0, 0)),
    ]
    args = [x, stats, w_bf]
    if res is not None:
        in_specs.append(pl.BlockSpec((bm, C), lambda i: (i, 0)))
        args.append(res)
    return pl.pallas_call(
        functools.partial(_mm_kernel, bm=bm, nj=nj, with_res=res is not None),
        grid=(nb,),
        in_specs=in_specs,
        out_specs=pl.BlockSpec((nblk, bm, C), lambda i: (0, i, 0)),
        out_shape=jax.ShapeDtypeStruct((nblk, NPAD, C), jnp.float32),
    )(*args)


def _sc_body(y_hbm, idx_hbm, out_hbm,
             idxv0, idxv1, rows0, rows1, out0, out1,
             sr0, sr1, so0, so1, si0, si1):
    w = lax.axis_index("s") * 2 + lax.axis_index("c")
    base = w * PPT
    idxv = (idxv0, idxv1)
    rows = (rows0, rows1)
    outs = (out0, out1)
    srs = (sr0, sr1)
    sos = (so0, so1)
    sis = (si0, si1)

    def issue(g, b):
        for r in range(NSEG):
            pltpu.async_copy(y_hbm.at[idxv[b].at[r]],
                             rows[b].at[pl.ds(r * SEGLEN, SEGLEN)],
                             srs[b])

    def drain_rows(b):
        pltpu.make_async_copy(y_hbm.at[pl.ds(0, EPC)], rows[b],
                              srs[b]).wait()

    def drain_out(b):
        pltpu.make_async_copy(out_hbm.at[pl.ds(0, CPP)], outs[b],
                              sos[b]).wait()

    def drain_idx(b):
        pltpu.make_async_copy(idx_hbm.at[0, 0], idxv[b], sis[b]).wait()

    pltpu.sync_copy(idx_hbm.at[w, 0], idxv0)
    issue(0, 0)
    pltpu.sync_copy(idx_hbm.at[w, 1], idxv1)
    issue(1, 1)

    def process(gg, b):
        drain_rows(b)

        @pl.when(gg + 2 < NCHUNK)
        def _():
            pltpu.async_copy(idx_hbm.at[w, gg + 2], idxv[b], sis[b])

        @pl.when(gg >= 2)
        def _():
            drain_out(b)

        def point(pp, c2):
            e0 = pp * NENT
            for grp in range(8):
                sl = pl.ds(grp * 16, 16)
                acc = rows[b][e0, sl]
                for j in range(1, NENT):
                    acc = acc + rows[b][e0 + j, sl]
                outs[b][pp, sl] = acc
            return c2

        lax.fori_loop(0, CPP, point, 0)
        pltpu.async_copy(outs[b], out_hbm.at[pl.ds(base + gg * CPP, CPP)],
                         sos[b])

        @pl.when(gg + 2 < NCHUNK)
        def _():
            drain_idx(b)
            issue(gg + 2, b)

    def chunk(g2, c):
        process(g2 * 2, 0)
        process(g2 * 2 + 1, 1)
        return c

    lax.fori_loop(0, NCHUNK // 2, chunk, 0)
    drain_out(0)
    drain_out(1)


def _sc_gathersum(y, idx):
    yf = y.reshape(-1, C)
    mesh = plsc.VectorSubcoreMesh(core_axis_name="c", subcore_axis_name="s")
    k = pl.kernel(
        _sc_body,
        out_type=jax.ShapeDtypeStruct((NPAD, C), jnp.float32),
        mesh=mesh,
        scratch_types=[
            pltpu.VMEM((NSEG, SEGLEN), jnp.int32),
            pltpu.VMEM((NSEG, SEGLEN), jnp.int32),
            pltpu.VMEM((EPC, C), jnp.float32),
            pltpu.VMEM((EPC, C), jnp.float32),
            pltpu.VMEM((CPP, C), jnp.float32),
            pltpu.VMEM((CPP, C), jnp.float32),
            pltpu.SemaphoreType.DMA,
            pltpu.SemaphoreType.DMA,
            pltpu.SemaphoreType.DMA,
            pltpu.SemaphoreType.DMA,
            pltpu.SemaphoreType.DMA,
            pltpu.SemaphoreType.DMA,
        ],
    )
    return k(yf, idx)


def _build_indices(pos):
    p = pos.astype(jnp.int32) + 1
    key = p[:, 0] * (G * G) + p[:, 1] * G + p[:, 2]
    table = jnp.full((TBL,), N, dtype=jnp.int32)
    table = table.at[key].min(jnp.arange(N, dtype=jnp.int32))
    offs = []
    for dx in (-1, 0, 1):
        for dy in (-1, 0, 1):
            for dz in (-1, 0, 1):
                offs.append(dx * G * G + dy * G + dz)
    offs = jnp.array(offs, dtype=jnp.int32)
    nk = key[:, None] + offs[None, :]
    src = jnp.full((NPAD, NOFF), N, dtype=jnp.int32).at[:N].set(table[nk])
    joff = jnp.arange(NOFF, dtype=jnp.int32) * NPAD
    i = jnp.arange(NPAD, dtype=jnp.int32)
    zrow = N + (i[:, None] + jnp.arange(NOFF, dtype=jnp.int32) * 7) \
        % (NPAD - N)
    rowid = jnp.where(src < N, src, zrow) + joff[None, :]
    z28 = (N + (i * 13) % (NPAD - N))[:, None]
    res28 = (NOFF * NPAD + i)[:, None]
    idx1 = jnp.concatenate([rowid, z28], axis=1)
    idx2 = jnp.concatenate([rowid, res28], axis=1)
    shape = (NW, NCHUNK, NSEG, SEGLEN)
    return idx1.reshape(shape), idx2.reshape(shape)


def kernel(feat, pos, W1, W2, bn1_w, bn1_b, bn2_w, bn2_b):
    idx1, idx2 = _build_indices(pos)
    featp = jnp.pad(feat, ((0, NPAD - N), (0, 0)))
    w1_bf = W1.astype(jnp.bfloat16)
    w2_bf = W2.astype(jnp.bfloat16)

    st1 = _bn_stats(featp).at[2].set(bn1_w).at[3].set(bn1_b)
    y1 = _bn_relu_mm(featp, st1, w1_bf)
    c1 = _sc_gathersum(y1, idx1)

    st2 = _bn_stats(c1).at[2].set(bn2_w).at[3].set(bn2_b)
    y2 = _bn_relu_mm(c1, st2, w2_bf, res=featp)
    out = _sc_gathersum(y2, idx2)
    return out[:N]


# speedup vs baseline: 1.0060x; 1.0041x over previous
import functools

import jax
import jax.numpy as jnp
from jax import lax
from jax.experimental import pallas as pl
from jax.experimental.pallas import tpu as pltpu
from jax.experimental.pallas import tpu_sc as plsc

G = 128
EPS = 1e-4
N = 50000
C = 128
NOFF = 27
NENT = 28

NW = 32
CPP = 16
PPT = 1568
NPAD = NW * PPT
NCHUNK = PPT // CPP
EPC = CPP * NENT
NSEG = 4
SEGLEN = 112

TBL = 102 * G * G + 102


def _stats_kernel(x_ref, o_ref):
    i = pl.program_id(0)

    @pl.when(i == 0)
    def _():
        o_ref[...] = jnp.zeros_like(o_ref)

    xb = x_ref[...].astype(jnp.float32)
    o_ref[0:1, :] += jnp.sum(xb, axis=0, keepdims=True)
    o_ref[1:2, :] += jnp.sum(xb * xb, axis=0, keepdims=True)


def _bn_stats(x, bm=1024):
    nb = x.shape[0] // bm
    return pl.pallas_call(
        _stats_kernel,
        grid=(nb,),
        in_specs=[pl.BlockSpec((bm, C), lambda i: (i, 0))],
        out_specs=pl.BlockSpec((8, C), lambda i: (0, 0)),
        out_shape=jax.ShapeDtypeStruct((8, C), jnp.float32),
    )(x)


def _mm_kernel(x_ref, st_ref, w_ref, *rest, bm, nj, with_res):
    if with_res:
        res_ref, y_ref = rest
    else:
        (y_ref,) = rest
    i = pl.program_id(0)
    s = st_ref[0:1, :]
    s2 = st_ref[1:2, :]
    bw = st_ref[2:3, :]
    bb = st_ref[3:4, :]
    mean = s * (1.0 / N)
    var = s2 * (1.0 / N) - mean * mean
    rstd = lax.rsqrt(var + EPS)
    scale = rstd * bw
    shift = bb - mean * scale

    xb = x_ref[...].astype(jnp.float32)
    h = jnp.maximum(xb * scale + shift, 0.0)
    rows = i * bm + lax.broadcasted_iota(jnp.int32, (bm, 1), 0)
    h = jnp.where(rows < N, h, 0.0).astype(jnp.bfloat16)
    for j in range(nj):
        y_ref[j] = jnp.dot(h, w_ref[j], preferred_element_type=jnp.float32)
    if with_res:
        y_ref[nj] = res_ref[...]


def _bn_relu_mm(x, stats, w_bf, res=None, bm=1024):
    nb = x.shape[0] // bm
    nj = w_bf.shape[0]
    nblk = nj + (1 if res is not None else 0)
    in_specs = [
        pl.BlockSpec((bm, C), lambda i: (i, 0)),
        pl.BlockSpec((8, C), lambda i: (0, 0)),
        pl.BlockSpec((nj, C, C), lambda i: (0, 0, 0)),
    ]
    args = [x, stats, w_bf]
    if res is not None:
        in_specs.append(pl.BlockSpec((bm, C), lambda i: (i, 0)))
        args.append(res)
    return pl.pallas_call(
        functools.partial(_mm_kernel, bm=bm, nj=nj, with_res=res is not None),
        grid=(nb,),
        in_specs=in_specs,
        out_specs=pl.BlockSpec((nblk, bm, C), lambda i: (0, i, 0)),
        out_shape=jax.ShapeDtypeStruct((nblk, NPAD, C), jnp.float32),
    )(*args)


def _sc_body(y_hbm, idx_hbm, out_hbm,
             idxv0, idxv1, rows0, rows1, out0, out1,
             sr0, sr1, so0, so1, si0, si1):
    w = lax.axis_index("s") * 2 + lax.axis_index("c")
    base = w * PPT
    idxv = (idxv0, idxv1)
    rows = (rows0, rows1)
    outs = (out0, out1)
    srs = (sr0, sr1)
    sos = (so0, so1)
    sis = (si0, si1)

    def issue(g, b):
        for r in range(NSEG):
            pltpu.async_copy(y_hbm.at[idxv[b].at[r]],
                             rows[b].at[pl.ds(r * SEGLEN, SEGLEN)],
                             srs[b])

    def drain_rows(b):
        pltpu.make_async_copy(y_hbm.at[pl.ds(0, EPC)], rows[b],
                              srs[b]).wait()

    def drain_out(b):
        pltpu.make_async_copy(out_hbm.at[pl.ds(0, CPP)], outs[b],
                              sos[b]).wait()

    def drain_idx(b):
        pltpu.make_async_copy(idx_hbm.at[0, 0], idxv[b], sis[b]).wait()

    pltpu.sync_copy(idx_hbm.at[w, 0], idxv0)
    issue(0, 0)
    pltpu.sync_copy(idx_hbm.at[w, 1], idxv1)
    issue(1, 1)

    def process(gg, b):
        drain_rows(b)

        @pl.when(gg + 2 < NCHUNK)
        def _():
            pltpu.async_copy(idx_hbm.at[w, gg + 2], idxv[b], sis[b])

        @pl.when(gg >= 2)
        def _():
            drain_out(b)

        def point(pp, c2):
            e0 = pp * NENT
            for grp in range(8):
                sl = pl.ds(grp * 16, 16)
                acc = rows[b][e0, sl]
                for j in range(1, NENT):
                    acc = acc + rows[b][e0 + j, sl]
                outs[b][pp, sl] = acc
            return c2

        lax.fori_loop(0, CPP, point, 0)
        pltpu.async_copy(outs[b], out_hbm.at[pl.ds(base + gg * CPP, CPP)],
                         sos[b])

        @pl.when(gg + 2 < NCHUNK)
        def _():
            drain_idx(b)
            issue(gg + 2, b)

    def chunk(g2, c):
        process(g2 * 2, 0)
        process(g2 * 2 + 1, 1)
        return c

    lax.fori_loop(0, NCHUNK // 2, chunk, 0)
    drain_out(0)
    drain_out(1)


def _sc_gathersum(y, idx):
    yf = y.reshape(-1, C)
    mesh = plsc.VectorSubcoreMesh(core_axis_name="c", subcore_axis_name="s")
    k = pl.kernel(
        _sc_body,
        out_type=jax.ShapeDtypeStruct((NPAD, C), jnp.float32),
        mesh=mesh,
        scratch_types=[
            pltpu.VMEM((NSEG, SEGLEN), jnp.int32),
            pltpu.VMEM((NSEG, SEGLEN), jnp.int32),
            pltpu.VMEM((EPC, C), jnp.float32),
            pltpu.VMEM((EPC, C), jnp.float32),
            pltpu.VMEM((CPP, C), jnp.float32),
            pltpu.VMEM((CPP, C), jnp.float32),
            pltpu.SemaphoreType.DMA,
            pltpu.SemaphoreType.DMA,
            pltpu.SemaphoreType.DMA,
            pltpu.SemaphoreType.DMA,
            pltpu.SemaphoreType.DMA,
            pltpu.SemaphoreType.DMA,
        ],
    )
    return k(yf, idx)


def _build_indices(pos):
    p = pos.astype(jnp.int32) + 1
    key = p[:, 0] * (G * G) + p[:, 1] * G + p[:, 2]
    table = jnp.full((TBL,), N, dtype=jnp.int32)
    table = table.at[key].min(jnp.arange(N, dtype=jnp.int32))
    offs = []
    for dx in (-1, 0, 1):
        for dy in (-1, 0, 1):
            for dz in (-1, 0, 1):
                offs.append(dx * G * G + dy * G + dz)
    offs = jnp.array(offs, dtype=jnp.int32)
    nk = key[:, None] + offs[None, :]
    src = jnp.full((NPAD, NOFF), N, dtype=jnp.int32).at[:N].set(table[nk])
    joff = jnp.arange(NOFF, dtype=jnp.int32) * NPAD
    i = jnp.arange(NPAD, dtype=jnp.int32)
    zrow = N + (i[:, None] + jnp.arange(NOFF, dtype=jnp.int32) * 7) \
        % (NPAD - N)
    rowid = jnp.where(src < N, src, zrow) + joff[None, :]
    z28 = (N + (i * 13) % (NPAD - N))[:, None]
    res28 = (NOFF * NPAD + i)[:, None]
    idx1 = jnp.concatenate([rowid, z28], axis=1)
    idx2 = jnp.concatenate([rowid, res28], axis=1)
    shape = (NW, NCHUNK, NSEG, SEGLEN)
    return idx1.reshape(shape), idx2.reshape(shape)


def kernel(feat, pos, W1, W2, bn1_w, bn1_b, bn2_w, bn2_b):
    idx1, idx2 = _build_indices(pos)
    featp = jnp.pad(feat, ((0, NPAD - N), (0, 0)))
    w1_bf = W1.astype(jnp.bfloat16)
    w2_bf = W2.astype(jnp.bfloat16)

    st1 = _bn_stats(featp).at[2].set(bn1_w).at[3].set(bn1_b)
    y1 = _bn_relu_mm(featp, st1, w1_bf)
    c1 = _sc_gathersum(y1, idx1)

    st2 = _bn_stats(c1).at[2].set(bn2_w).at[3].set(bn2_b)
    y2 = _bn_relu_mm(c1, st2, w2_bf, res=featp)
    out = _sc_gathersum(y2, idx2)
    return out[:N]
